# baseline (speedup 1.0000x reference)
import functools

import jax
import jax.numpy as jnp
import numpy as np
from jax import lax
from jax.experimental import pallas as pl
from jax.experimental.pallas import tpu as pltpu
from jax.experimental.pallas import tpu_sc as plsc

_THRESHOLD = 0.05
_CACHELINE = 64
_VBITS = 24
_NWORKERS = 32
_W = 128
_G = _W // 16


def _make_sc_call(n_chunks):
    assert n_chunks % _W == 0
    n_windows = n_chunks // _W
    trips = -(-n_windows // _NWORKERS)
    mesh = plsc.VectorSubcoreMesh(core_axis_name="c", subcore_axis_name="s")

    @functools.partial(
        pl.kernel,
        mesh=mesh,
        out_type=jax.ShapeDtypeStruct((_CACHELINE, n_chunks), jnp.float32),
        scratch_types=[
            pltpu.VMEM((_CACHELINE, _W), jnp.float32),
            pltpu.VMEM((_CACHELINE, _W), jnp.float32),
            pltpu.VMEM((_CACHELINE, 16), jnp.float32),
            pltpu.VMEM((_CACHELINE, 16), jnp.float32),
        ],
    )
    def sc_cluster(xt_hbm, ot_hbm, xv, ov, bv, pk):
        wid = lax.axis_index("s") * 2 + lax.axis_index("c")
        inf = jnp.full((16,), jnp.inf, jnp.float32)

        def window_body(b, carry):
            widx = b * _NWORKERS + wid

            @pl.when(widx < n_windows)
            def _():
                _do_window(widx)

            return carry

        def _do_window(widx):
            c0 = widx * _W
            pltpu.sync_copy(xt_hbm.at[:, pl.ds(c0, _W)], xv)

            def group_body(g, gcarry):
                sl = pl.ds(g * 16, 16)
                for i in range(_CACHELINE):
                    v = xv[i, sl]
                    vb = lax.bitcast_convert_type(v, jnp.int32)
                    pk_i = lax.bitcast_convert_type(
                        ((1 << 30) | (i << _VBITS))
                        | lax.shift_right_logical(vb, 32 - _VBITS),
                        jnp.float32,
                    )
                    if i == 0:
                        ov[i, sl] = v
                        bv[0] = v
                        pk[0] = pk_i
                        continue

                    def j_body(j, kmin):
                        m = jnp.abs(bv[j] - v) < _THRESHOLD
                        return jnp.minimum(kmin, jnp.where(m, pk[j], inf))

                    kmin = lax.fori_loop(0, i, j_body, inf, unroll=4)
                    found = kmin < inf
                    ki = lax.bitcast_convert_type(kmin, jnp.int32)
                    mv = lax.bitcast_convert_type(
                        (ki & ((1 << _VBITS) - 1)) << (32 - _VBITS),
                        jnp.float32,
                    )
                    ov[i, sl] = jnp.where(found, mv, v)
                    bv[i] = jnp.where(found, inf, v)
                    pk[i] = pk_i
                return gcarry

            lax.fori_loop(0, _G, group_body, 0)
            pltpu.sync_copy(ov, ot_hbm.at[:, pl.ds(c0, _W)])

        lax.fori_loop(0, trips, window_body, 0)

    return sc_cluster


@functools.partial(jax.jit, static_argnums=(1,))
def _cluster_flat_sc(xt, n_chunks):
    return _make_sc_call(n_chunks)(xt)


def kernel(x):
    shape = x.shape
    flat = x.reshape(-1)
    total = flat.shape[0]
    n_full = (total // _CACHELINE) * _CACHELINE
    n_chunks = n_full // _CACHELINE
    xt = flat[:n_full].reshape(n_chunks, _CACHELINE).T
    out_t = _cluster_flat_sc(xt, n_chunks)
    out = out_t.T.reshape(-1)
    if n_full != total:
        out = jnp.concatenate([out, flat[n_full:]])
    return out.reshape(shape)


# speedup vs baseline: 1.6949x; 1.6949x over previous
import functools

import jax
import jax.numpy as jnp
import numpy as np
from jax import lax
from jax.experimental import pallas as pl
from jax.experimental.pallas import tpu as pltpu
from jax.experimental.pallas import tpu_sc as plsc

_THRESHOLD = 0.05
_CACHELINE = 64
_VBITS = 24
_SC_CHUNKS = 73728


def _cluster_body(x_ref, o_ref, xt_ref, bv_ref, pk_ref, ot_ref):
    inf = jnp.float32(jnp.inf)
    xt = x_ref[0].T
    xt_ref[:] = xt
    rows = jax.lax.broadcasted_iota(jnp.int32, xt.shape, 0)
    xbits = jax.lax.bitcast_convert_type(xt, jnp.int32)
    packed = ((1 << 30) | (rows << _VBITS)) | jax.lax.shift_right_logical(
        xbits, 32 - _VBITS
    )
    pk_ref[:] = jax.lax.bitcast_convert_type(packed, jnp.float32)
    bv_ref[:] = jnp.full(bv_ref.shape, inf, jnp.float32)
    lanes = bv_ref.shape[1]
    half = lanes // 2

    def step(i, lo, hi):
        v = xt_ref[i : i + 1, lo:hi]
        if i == 0:
            ot_ref[i : i + 1, lo:hi] = v
            bv_ref[i : i + 1, lo:hi] = v
            return
        nrows = min(-(-i // 8) * 8, _CACHELINE)
        m = jnp.abs(bv_ref[0:nrows, lo:hi] - v) < _THRESHOLD
        key = jnp.where(m, pk_ref[0:nrows, lo:hi], inf)
        kmin = jnp.min(key, axis=0, keepdims=True)
        found = kmin < inf
        ki = jax.lax.bitcast_convert_type(kmin, jnp.int32)
        mv_bits = (ki & ((1 << _VBITS) - 1)) << (32 - _VBITS)
        mv = jax.lax.bitcast_convert_type(mv_bits, jnp.float32)
        ot_ref[i : i + 1, lo:hi] = jnp.where(found, mv, v)
        bv_ref[i : i + 1, lo:hi] = jnp.where(found, inf, v)

    for i in range(_CACHELINE):
        step(i, 0, half)
        step(i, half, lanes)
    o_ref[0] = ot_ref[:].T


def _cluster_flat_tc(xc, lanes):
    n = xc.shape[0]
    grid = n // lanes
    x3 = xc.reshape(grid, lanes, _CACHELINE)
    out = pl.pallas_call(
        _cluster_body,
        grid=(grid,),
        in_specs=[pl.BlockSpec((1, lanes, _CACHELINE), lambda i: (i, 0, 0))],
        out_specs=pl.BlockSpec((1, lanes, _CACHELINE), lambda i: (i, 0, 0)),
        out_shape=jax.ShapeDtypeStruct((grid, lanes, _CACHELINE), jnp.float32),
        scratch_shapes=[
            pltpu.VMEM((_CACHELINE, lanes), jnp.float32),
            pltpu.VMEM((_CACHELINE, lanes), jnp.float32),
            pltpu.VMEM((_CACHELINE, lanes), jnp.float32),
            pltpu.VMEM((_CACHELINE, lanes), jnp.float32),
        ],
    )(x3)
    return out.reshape(n, _CACHELINE)


_NWORKERS = 32
_W = 128
_G = _W // 16


def _make_sc_call(n_chunks):
    assert n_chunks % _W == 0
    n_windows = n_chunks // _W
    trips = -(-n_windows // _NWORKERS)
    mesh = plsc.VectorSubcoreMesh(core_axis_name="c", subcore_axis_name="s")

    @functools.partial(
        pl.kernel,
        mesh=mesh,
        out_type=jax.ShapeDtypeStruct((_CACHELINE, n_chunks), jnp.float32),
        scratch_types=[
            pltpu.VMEM((_CACHELINE, _W), jnp.float32),
            pltpu.VMEM((_CACHELINE, _W), jnp.float32),
            pltpu.VMEM((_CACHELINE, 16), jnp.float32),
            pltpu.VMEM((_CACHELINE, 16), jnp.float32),
        ],
    )
    def sc_cluster(xt_hbm, ot_hbm, xv, ov, bv, pk):
        wid = lax.axis_index("s") * 2 + lax.axis_index("c")
        inf = jnp.full((16,), jnp.inf, jnp.float32)

        def window_body(b, carry):
            widx = b * _NWORKERS + wid

            @pl.when(widx < n_windows)
            def _():
                _do_window(widx)

            return carry

        def _do_window(widx):
            c0 = widx * _W
            pltpu.sync_copy(xt_hbm.at[:, pl.ds(c0, _W)], xv)

            def group_body(g, gcarry):
                sl = pl.ds(g * 16, 16)
                for i in range(_CACHELINE):
                    v = xv[i, sl]
                    vb = lax.bitcast_convert_type(v, jnp.int32)
                    pk_i = lax.bitcast_convert_type(
                        ((1 << 30) | (i << _VBITS))
                        | lax.shift_right_logical(vb, 32 - _VBITS),
                        jnp.float32,
                    )
                    if i == 0:
                        ov[i, sl] = v
                        bv[0] = v
                        pk[0] = pk_i
                        continue

                    def j_body(j, kmin):
                        m = jnp.abs(bv[j] - v) < _THRESHOLD
                        return jnp.minimum(kmin, jnp.where(m, pk[j], inf))

                    kmin = lax.fori_loop(0, i, j_body, inf, unroll=4)
                    found = kmin < inf
                    ki = lax.bitcast_convert_type(kmin, jnp.int32)
                    mv = lax.bitcast_convert_type(
                        (ki & ((1 << _VBITS) - 1)) << (32 - _VBITS),
                        jnp.float32,
                    )
                    ov[i, sl] = jnp.where(found, mv, v)
                    bv[i] = jnp.where(found, inf, v)
                    pk[i] = pk_i
                return gcarry

            lax.fori_loop(0, _G, group_body, 0)
            pltpu.sync_copy(ov, ot_hbm.at[:, pl.ds(c0, _W)])

        lax.fori_loop(0, trips, window_body, 0)

    return sc_cluster


@functools.partial(jax.jit, static_argnums=(1, 2))
def _cluster_split(flat, n_chunks, sc_chunks):
    ka = n_chunks - sc_chunks
    out_a = _cluster_flat_tc(flat[: ka * _CACHELINE].reshape(ka, _CACHELINE),
                             1024)
    xt_b = flat[ka * _CACHELINE :].reshape(sc_chunks, _CACHELINE).T
    ot_b = _make_sc_call(sc_chunks)(xt_b)
    return jnp.concatenate(
        [out_a.reshape(-1), ot_b.T.reshape(-1)]
    )


def kernel(x):
    shape = x.shape
    flat = x.reshape(-1)
    total = flat.shape[0]
    n_full = (total // _CACHELINE) * _CACHELINE
    n_chunks = n_full // _CACHELINE
    sc_chunks = _SC_CHUNKS
    if (n_chunks - sc_chunks) % 1024 != 0 or sc_chunks > n_chunks:
        sc_chunks = 0
    if sc_chunks:
        out = _cluster_split(flat[:n_full], n_chunks, sc_chunks)
    else:
        lanes = next(c for c in (1024, 512, 256, 128, 64, 32, 16, 8, 4, 2, 1)
                     if n_chunks % c == 0)
        out = _cluster_flat_tc(
            flat[:n_full].reshape(n_chunks, _CACHELINE), lanes
        ).reshape(-1)
    if n_full != total:
        out = jnp.concatenate([out, flat[n_full:]])
    return out.reshape(shape)


# speedup vs baseline: 4.5037x; 2.6573x over previous
import functools

import jax
import jax.numpy as jnp
import numpy as np
from jax.experimental import pallas as pl
from jax.experimental.pallas import tpu as pltpu

_THRESHOLD = 0.05
_CACHELINE = 64
_VBITS = 24
_LANES = 1024
_NG = _LANES // 128


def _cluster_body(x_ref, o_ref, xt3, bv3, pk3, ot3):
    inf = jnp.float32(jnp.inf)
    for g in range(_NG):
        xt3[:, g, :] = x_ref[0, g * 128 : (g + 1) * 128, :].T
    xt_all = xt3[:]
    rows = jax.lax.broadcasted_iota(jnp.int32, xt_all.shape, 0)
    xbits = jax.lax.bitcast_convert_type(xt_all, jnp.int32)
    packed = ((1 << 30) | (rows << _VBITS)) | jax.lax.shift_right_logical(
        xbits, 32 - _VBITS
    )
    pk3[:] = jax.lax.bitcast_convert_type(packed, jnp.float32)
    inf_t = jnp.full((8, 128), inf, jnp.float32)
    for i in range(_CACHELINE):
        v = xt3[i]
        if i == 0:
            ot3[0] = v
            bv3[0] = v
            continue
        acc = [inf_t, inf_t, inf_t, inf_t]
        for j in range(i):
            m = jnp.abs(bv3[j] - v) < _THRESHOLD
            acc[j % 4] = jnp.minimum(acc[j % 4], jnp.where(m, pk3[j], inf))
        kmin = jnp.minimum(
            jnp.minimum(acc[0], acc[1]), jnp.minimum(acc[2], acc[3])
        )
        found = kmin < inf
        ki = jax.lax.bitcast_convert_type(kmin, jnp.int32)
        mv_bits = (ki & ((1 << _VBITS) - 1)) << (32 - _VBITS)
        mv = jax.lax.bitcast_convert_type(mv_bits, jnp.float32)
        ot3[i] = jnp.where(found, mv, v)
        bv3[i] = jnp.where(found, inf, v)
    for g in range(_NG):
        o_ref[0, g * 128 : (g + 1) * 128, :] = ot3[:, g, :].T


def _cluster_flat_tc(xc):
    n = xc.shape[0]
    grid = n // _LANES
    x3 = xc.reshape(grid, _LANES, _CACHELINE)
    out = pl.pallas_call(
        _cluster_body,
        grid=(grid,),
        in_specs=[pl.BlockSpec((1, _LANES, _CACHELINE), lambda i: (i, 0, 0))],
        out_specs=pl.BlockSpec((1, _LANES, _CACHELINE), lambda i: (i, 0, 0)),
        out_shape=jax.ShapeDtypeStruct((grid, _LANES, _CACHELINE), jnp.float32),
        scratch_shapes=[
            pltpu.VMEM((_CACHELINE, _NG, 128), jnp.float32),
            pltpu.VMEM((_CACHELINE, _NG, 128), jnp.float32),
            pltpu.VMEM((_CACHELINE, _NG, 128), jnp.float32),
            pltpu.VMEM((_CACHELINE, _NG, 128), jnp.float32),
        ],
    )(x3)
    return out.reshape(n, _CACHELINE)


@jax.jit
def _cluster(flat):
    n_chunks = flat.shape[0] // _CACHELINE
    return _cluster_flat_tc(flat.reshape(n_chunks, _CACHELINE)).reshape(-1)


def kernel(x):
    shape = x.shape
    flat = x.reshape(-1)
    total = flat.shape[0]
    n_full = (total // _CACHELINE) * _CACHELINE
    n_chunks = n_full // _CACHELINE
    assert n_chunks % _LANES == 0
    out = _cluster(flat[:n_full])
    if n_full != total:
        out = jnp.concatenate([out, flat[n_full:]])
    return out.reshape(shape)


# speedup vs baseline: 4.6516x; 1.0328x over previous
import functools

import jax
import jax.numpy as jnp
import numpy as np
from jax.experimental import pallas as pl
from jax.experimental.pallas import tpu as pltpu

_THRESHOLD = 0.05
_CACHELINE = 64
_VBITS = 24
_LANES = 2048
_NG = _LANES // 128


def _cluster_body(x_ref, o_ref, xt3, bv3, pk3, ot3):
    inf = jnp.float32(jnp.inf)
    for g in range(_NG):
        xt3[:, g, :] = x_ref[0, g * 128 : (g + 1) * 128, :].T
    xt_all = xt3[:]
    rows = jax.lax.broadcasted_iota(jnp.int32, xt_all.shape, 0)
    xbits = jax.lax.bitcast_convert_type(xt_all, jnp.int32)
    packed = ((1 << 30) | (rows << _VBITS)) | jax.lax.shift_right_logical(
        xbits, 32 - _VBITS
    )
    pk3[:] = jax.lax.bitcast_convert_type(packed, jnp.float32)
    inf_t = jnp.full((_NG, 128), inf, jnp.float32)
    for i in range(_CACHELINE):
        v = xt3[i]
        if i == 0:
            ot3[0] = v
            bv3[0] = v
            continue
        acc = [inf_t, inf_t, inf_t, inf_t]
        for j in range(i):
            m = jnp.abs(bv3[j] - v) < _THRESHOLD
            acc[j % 4] = jnp.minimum(acc[j % 4], jnp.where(m, pk3[j], inf))
        kmin = jnp.minimum(
            jnp.minimum(acc[0], acc[1]), jnp.minimum(acc[2], acc[3])
        )
        found = kmin < inf
        ki = jax.lax.bitcast_convert_type(kmin, jnp.int32)
        mv_bits = (ki & ((1 << _VBITS) - 1)) << (32 - _VBITS)
        mv = jax.lax.bitcast_convert_type(mv_bits, jnp.float32)
        ot3[i] = jnp.where(found, mv, v)
        bv3[i] = jnp.where(found, inf, v)
    for g in range(_NG):
        o_ref[0, g * 128 : (g + 1) * 128, :] = ot3[:, g, :].T


def _cluster_flat_tc(xc):
    n = xc.shape[0]
    grid = n // _LANES
    x3 = xc.reshape(grid, _LANES, _CACHELINE)
    out = pl.pallas_call(
        _cluster_body,
        grid=(grid,),
        in_specs=[pl.BlockSpec((1, _LANES, _CACHELINE), lambda i: (i, 0, 0))],
        out_specs=pl.BlockSpec((1, _LANES, _CACHELINE), lambda i: (i, 0, 0)),
        out_shape=jax.ShapeDtypeStruct((grid, _LANES, _CACHELINE), jnp.float32),
        scratch_shapes=[
            pltpu.VMEM((_CACHELINE, _NG, 128), jnp.float32),
            pltpu.VMEM((_CACHELINE, _NG, 128), jnp.float32),
            pltpu.VMEM((_CACHELINE, _NG, 128), jnp.float32),
            pltpu.VMEM((_CACHELINE, _NG, 128), jnp.float32),
        ],
    )(x3)
    return out.reshape(n, _CACHELINE)


@jax.jit
def _cluster(flat):
    n_chunks = flat.shape[0] // _CACHELINE
    return _cluster_flat_tc(flat.reshape(n_chunks, _CACHELINE)).reshape(-1)


def kernel(x):
    shape = x.shape
    flat = x.reshape(-1)
    total = flat.shape[0]
    n_full = (total // _CACHELINE) * _CACHELINE
    n_chunks = n_full // _CACHELINE
    assert n_chunks % _LANES == 0
    out = _cluster(flat[:n_full])
    if n_full != total:
        out = jnp.concatenate([out, flat[n_full:]])
    return out.reshape(shape)


# speedup vs baseline: 4.6547x; 1.0007x over previous
import jax
import jax.numpy as jnp
from jax.experimental import pallas as pl
from jax.experimental.pallas import tpu as pltpu

_THRESHOLD = 0.05
_CACHELINE = 64
_VBITS = 24
_LANES = 2048
_NG = _LANES // 128


def _cluster_body(x_ref, o_ref, xt3, bv3, pk3, ot3):
    inf = jnp.float32(jnp.inf)
    for g in range(_NG):
        xt3[:, g, :] = x_ref[0, g * 128 : (g + 1) * 128, :].T
    xt_all = xt3[:]
    rows = jax.lax.broadcasted_iota(jnp.int32, xt_all.shape, 0)
    xbits = jax.lax.bitcast_convert_type(xt_all, jnp.int32)
    packed = ((1 << 30) | (rows << _VBITS)) | jax.lax.shift_right_logical(
        xbits, 32 - _VBITS
    )
    pk3[:] = jax.lax.bitcast_convert_type(packed, jnp.float32)
    inf_t = jnp.full((_NG, 128), inf, jnp.float32)
    for i in range(_CACHELINE):
        v = xt3[i]
        if i == 0:
            ot3[0] = v
            bv3[0] = v
            continue
        acc = [inf_t, inf_t, inf_t, inf_t]
        for j in range(i):
            m = jnp.abs(bv3[j] - v) < _THRESHOLD
            acc[j % 4] = jnp.minimum(acc[j % 4], jnp.where(m, pk3[j], inf))
        kmin = jnp.minimum(
            jnp.minimum(acc[0], acc[1]), jnp.minimum(acc[2], acc[3])
        )
        found = kmin < inf
        ki = jax.lax.bitcast_convert_type(kmin, jnp.int32)
        mv_bits = (ki & ((1 << _VBITS) - 1)) << (32 - _VBITS)
        mv = jax.lax.bitcast_convert_type(mv_bits, jnp.float32)
        ot3[i] = jnp.where(found, mv, v)
        bv3[i] = jnp.where(found, inf, v)
    for g in range(_NG):
        o_ref[0, g * 128 : (g + 1) * 128, :] = ot3[:, g, :].T


def _cluster_flat_tc(xc):
    n = xc.shape[0]
    grid = n // _LANES
    x3 = xc.reshape(grid, _LANES, _CACHELINE)
    out = pl.pallas_call(
        _cluster_body,
        grid=(grid,),
        in_specs=[pl.BlockSpec((1, _LANES, _CACHELINE), lambda i: (i, 0, 0))],
        out_specs=pl.BlockSpec((1, _LANES, _CACHELINE), lambda i: (i, 0, 0)),
        out_shape=jax.ShapeDtypeStruct((grid, _LANES, _CACHELINE), jnp.float32),
        scratch_shapes=[
            pltpu.VMEM((_CACHELINE, _NG, 128), jnp.float32),
            pltpu.VMEM((_CACHELINE, _NG, 128), jnp.float32),
            pltpu.VMEM((_CACHELINE, _NG, 128), jnp.float32),
            pltpu.VMEM((_CACHELINE, _NG, 128), jnp.float32),
        ],
    )(x3)
    return out.reshape(n, _CACHELINE)


@jax.jit
def _cluster(flat):
    n_chunks = flat.shape[0] // _CACHELINE
    return _cluster_flat_tc(flat.reshape(n_chunks, _CACHELINE)).reshape(-1)


def kernel(x):
    shape = x.shape
    flat = x.reshape(-1)
    total = flat.shape[0]
    n_full = (total // _CACHELINE) * _CACHELINE
    n_chunks = n_full // _CACHELINE
    assert n_chunks % _LANES == 0
    out = _cluster(flat[:n_full])
    if n_full != total:
        out = jnp.concatenate([out, flat[n_full:]])
    return out.reshape(shape)
